# baseline (speedup 1.0000x reference)
import functools
import math

import jax
import jax.numpy as jnp
from jax import lax
from jax.experimental import pallas as pl
from jax.experimental.pallas import tpu as pltpu
from jax.experimental.pallas import tpu_sc as plsc

_NC = 2
_NS = 16
_NW = _NC * _NS
_L = 16

_VOCAB = 1000000
_EMB = 64
_SCALE = math.sqrt(_EMB)

_VT = 128
_N_BLOCKS = _VOCAB // _VT
_TAIL = _VOCAB - _N_BLOCKS * _VT
_BLOCKS_PER_W = (_N_BLOCKS + _NW - 1) // _NW


def _detile_body(tab_hbm, staged_hbm, bufs, isems, osems):
    wid = lax.axis_index("s") * _NC + lax.axis_index("c")

    def valid(j):
        return (j * _NW + wid) < _N_BLOCKS

    def in_copy(b, j):
        tile = j * _NW + wid
        return pltpu.make_async_copy(
            tab_hbm.at[:, :, pl.ds(pl.multiple_of(tile * _VT, _VT), _VT)],
            bufs.at[b], isems[b])

    def out_copy(b, j):
        return pltpu.make_async_copy(
            bufs.at[b], staged_hbm.at[j * _NW + wid], osems[b])

    for b in range(4):
        @pl.when(valid(b))
        def _():
            in_copy(b, b).start()

    @pl.loop(0, _BLOCKS_PER_W)
    def _(i):
        for b in range(4):
            @pl.when((lax.rem(i, 4) == b) & valid(i))
            def _():
                in_copy(b, i).wait()
                out_copy(b, i).start()
        @pl.when((i >= 3) & (i + 1 < _BLOCKS_PER_W))
        def _():
            for b in range(4):
                @pl.when((lax.rem(i + 1, 4) == b) & valid(i - 3))
                def _():
                    out_copy(b, i - 3).wait()
                    @pl.when(valid(i + 1))
                    def _():
                        in_copy(b, i + 1).start()

    for d in range(4):
        i = _BLOCKS_PER_W - 4 + d
        for b in range(4):
            @pl.when((lax.rem(i, 4) == b) & valid(i))
            def _():
                out_copy(b, i).wait()


def _transpose_body(staged_hbm, tail_hbm, lin_hbm, src, dst, tvm,
                    isems, osems):
    wid = lax.axis_index("s") * _NC + lax.axis_index("c")

    lane = lax.iota(jnp.int32, _L)
    eg = [lane + (c * _L) for c in range(4)]
    i0 = [jnp.right_shift(g, 3) for g in eg]
    i1 = [jnp.bitwise_and(g, 7) for g in eg]

    def valid(j):
        return (j * _NW + wid) < _N_BLOCKS

    def in_copy(b, j):
        return pltpu.make_async_copy(
            staged_hbm.at[j * _NW + wid], src.at[b], isems[b])

    def transpose(b):
        @pl.loop(0, _VT, unroll=4)
        def _(v):
            iv = jnp.full((_L,), v, dtype=jnp.int32)
            for c in range(4):
                vals = plsc.load_gather(src.at[b], [i0[c], i1[c], iv])
                dst[b, v, pl.ds(c * _L, _L)] = vals

    def out_copy(b, j):
        tile = j * _NW + wid
        return pltpu.make_async_copy(
            dst.at[b], lin_hbm.at[pl.ds(tile * _VT, _VT)], osems[b])

    @pl.when(wid == 0)
    def _():
        pltpu.sync_copy(tail_hbm, tvm)
        pltpu.sync_copy(tvm, lin_hbm.at[pl.ds(_N_BLOCKS * _VT, _TAIL)])

    for b in range(2):
        @pl.when(valid(b))
        def _():
            in_copy(b, b).start()

    @pl.loop(0, _BLOCKS_PER_W)
    def _(j):
        for bb in range(2):
            @pl.when(lax.rem(j, 2) == bb)
            def _():
                @pl.when((j >= 2) & valid(j - 2))
                def _():
                    out_copy(bb, j - 2).wait()
                @pl.when(valid(j))
                def _():
                    in_copy(bb, j).wait()
                    transpose(bb)
                    out_copy(bb, j).start()
                    @pl.when(valid(j + 2) & (j + 2 < _BLOCKS_PER_W))
                    def _():
                        in_copy(bb, j + 2).start()

    for bb in range(2):
        j_last = _BLOCKS_PER_W - 2 + bb
        @pl.when(valid(j_last))
        def _():
            out_copy(bb, j_last).wait()


def _gather_body(tokt_hbm, lin_hbm, p_hbm, idx_v, gbuf, tbuf, gsems, osems,
                 *, t_per_w, b_per_w):
    wid = lax.axis_index("s") * _NC + lax.axis_index("c")
    tg = wid // 4
    bg = lax.rem(wid, 4)
    nbt = b_per_w // _VT

    pltpu.sync_copy(
        tokt_hbm.at[pl.ds(tg * t_per_w, t_per_w), pl.ds(bg * b_per_w, b_per_w)],
        idx_v)

    lane = lax.iota(jnp.int32, _L)
    eg = [lane + (c * _L) for c in range(4)]
    i0 = [jnp.right_shift(g, 3) for g in eg]
    i2 = [jnp.bitwise_and(g, 7) for g in eg]
    zero = jnp.zeros((_L,), dtype=jnp.int32)

    def gather(k, s):
        return pltpu.make_async_copy(
            lin_hbm.at[idx_v.at[s, pl.ds(k * _VT, _VT)]],
            gbuf.at[pl.ds(k * _VT, _VT)], gsems[k])

    def out_copy(tb, s, k):
        return pltpu.make_async_copy(
            tbuf.at[tb],
            p_hbm.at[tg * t_per_w + s, :, pl.ds(bg * nbt + k, 1)],
            osems[tb])

    def transpose_block(tb, k):
        @pl.loop(0, _VT, unroll=4)
        def _(b):
            ib = jnp.full((_L,), b, dtype=jnp.int32)
            for c in range(4):
                vals = gbuf[k * _VT + b, pl.ds(c * _L, _L)] * _SCALE
                plsc.store_scatter(tbuf.at[tb], [i0[c], zero, i2[c], ib], vals)

    @pl.loop(0, t_per_w)
    def _(s):
        for k in range(8):
            gather(k, s).start()
        for k in range(8):
            gather(k, s).wait()
            tb = k % 2
            pk = (k - 2) % 8
            if k >= 2:
                out_copy(tb, s, pk).wait()
            else:
                @pl.when(s > 0)
                def _():
                    out_copy(tb, s - 1, pk).wait()
            transpose_block(tb, k)
            out_copy(tb, s, k).start()

    for k in range(6, 8):
        out_copy(k % 2, t_per_w - 1, k).wait()


def kernel(tokens, table):
    bsz, seq = tokens.shape
    vocab, emb = table.shape
    assert (vocab, emb) == (_VOCAB, _EMB) and (bsz, seq) == (4096, 200)
    t_per_w = seq // 8
    b_per_w = bsz // 4

    tokens = tokens.astype(jnp.int32)
    table = table.astype(jnp.float32)

    mesh = plsc.VectorSubcoreMesh(
        core_axis_name="c", subcore_axis_name="s",
        num_cores=_NC, num_subcores=_NS)

    lin2 = table

    tokt = tokens.T
    body = functools.partial(_gather_body, t_per_w=t_per_w, b_per_w=b_per_w)
    p = pl.kernel(
        body,
        out_type=jax.ShapeDtypeStruct((seq, 8, 32, 8, _VT), jnp.float32),
        mesh=mesh,
        compiler_params=pltpu.CompilerParams(use_tc_tiling_on_sc=False,
                                             needs_layout_passes=False),
        scratch_types=dict(
            idx_v=pltpu.VMEM((t_per_w, b_per_w), jnp.int32),
            gbuf=pltpu.VMEM((b_per_w, _EMB), jnp.float32),
            tbuf=pltpu.VMEM((2, 8, 1, 8, _VT), jnp.float32),
            gsems=[pltpu.SemaphoreType.DMA] * 8,
            osems=[pltpu.SemaphoreType.DMA] * 2,
        ),
    )(tokt, lin2)
    out = p.transpose(2, 4, 0, 1, 3).reshape(bsz, seq, emb)
    return out


# speedup vs baseline: 1.3011x; 1.3011x over previous
import functools
import math

import jax
import jax.numpy as jnp
from jax import lax
from jax.experimental import pallas as pl
from jax.experimental.pallas import tpu as pltpu
from jax.experimental.pallas import tpu_sc as plsc

_NC = 2
_NS = 16
_NW = _NC * _NS
_L = 16

_VOCAB = 1000000
_EMB = 64
_SCALE = math.sqrt(_EMB)

_VT = 128
_N_BLOCKS = _VOCAB // _VT
_TAIL = _VOCAB - _N_BLOCKS * _VT
_BLOCKS_PER_W = (_N_BLOCKS + _NW - 1) // _NW


def _detile_body(tab_hbm, staged_hbm, bufs, isems, osems):
    wid = lax.axis_index("s") * _NC + lax.axis_index("c")

    def valid(j):
        return (j * _NW + wid) < _N_BLOCKS

    def in_copy(b, j):
        tile = j * _NW + wid
        return pltpu.make_async_copy(
            tab_hbm.at[:, :, pl.ds(pl.multiple_of(tile * _VT, _VT), _VT)],
            bufs.at[b], isems[b])

    def out_copy(b, j):
        return pltpu.make_async_copy(
            bufs.at[b], staged_hbm.at[j * _NW + wid], osems[b])

    for b in range(4):
        @pl.when(valid(b))
        def _():
            in_copy(b, b).start()

    @pl.loop(0, _BLOCKS_PER_W)
    def _(i):
        for b in range(4):
            @pl.when((lax.rem(i, 4) == b) & valid(i))
            def _():
                in_copy(b, i).wait()
                out_copy(b, i).start()
        @pl.when((i >= 3) & (i + 1 < _BLOCKS_PER_W))
        def _():
            for b in range(4):
                @pl.when((lax.rem(i + 1, 4) == b) & valid(i - 3))
                def _():
                    out_copy(b, i - 3).wait()
                    @pl.when(valid(i + 1))
                    def _():
                        in_copy(b, i + 1).start()

    for d in range(4):
        i = _BLOCKS_PER_W - 4 + d
        for b in range(4):
            @pl.when((lax.rem(i, 4) == b) & valid(i))
            def _():
                out_copy(b, i).wait()


def _transpose_body(staged_hbm, tail_hbm, lin_hbm, src, dst, tvm,
                    isems, osems):
    wid = lax.axis_index("s") * _NC + lax.axis_index("c")

    lane = lax.iota(jnp.int32, _L)
    eg = [lane + (c * _L) for c in range(4)]
    i0 = [jnp.right_shift(g, 3) for g in eg]
    i1 = [jnp.bitwise_and(g, 7) for g in eg]

    def valid(j):
        return (j * _NW + wid) < _N_BLOCKS

    def in_copy(b, j):
        return pltpu.make_async_copy(
            staged_hbm.at[j * _NW + wid], src.at[b], isems[b])

    def transpose(b):
        @plsc.parallel_loop(0, _VT, 1, unroll=4)
        def _(v):
            iv = jnp.full((_L,), v, dtype=jnp.int32)
            for c in range(4):
                vals = plsc.load_gather(src.at[b], [i0[c], i1[c], iv])
                dst[b, v, pl.ds(c * _L, _L)] = vals

    def out_copy(b, j):
        tile = j * _NW + wid
        return pltpu.make_async_copy(
            dst.at[b], lin_hbm.at[pl.ds(tile * _VT, _VT)], osems[b])

    @pl.when(wid == 0)
    def _():
        pltpu.sync_copy(tail_hbm, tvm)
        pltpu.sync_copy(tvm, lin_hbm.at[pl.ds(_N_BLOCKS * _VT, _TAIL)])

    for b in range(2):
        @pl.when(valid(b))
        def _():
            in_copy(b, b).start()

    @pl.loop(0, _BLOCKS_PER_W)
    def _(j):
        for bb in range(2):
            @pl.when(lax.rem(j, 2) == bb)
            def _():
                @pl.when((j >= 2) & valid(j - 2))
                def _():
                    out_copy(bb, j - 2).wait()
                @pl.when(valid(j))
                def _():
                    in_copy(bb, j).wait()
                    transpose(bb)
                    out_copy(bb, j).start()
                    @pl.when(valid(j + 2) & (j + 2 < _BLOCKS_PER_W))
                    def _():
                        in_copy(bb, j + 2).start()

    for bb in range(2):
        j_last = _BLOCKS_PER_W - 2 + bb
        @pl.when(valid(j_last))
        def _():
            out_copy(bb, j_last).wait()


def _gather_body(tokt_hbm, lin_hbm, p_hbm, idx_v, gbuf, tbuf, gsems, osems,
                 *, t_per_w, b_per_w):
    wid = lax.axis_index("s") * _NC + lax.axis_index("c")
    tg = wid // 4
    bg = lax.rem(wid, 4)
    nbt = b_per_w // _VT

    pltpu.sync_copy(
        tokt_hbm.at[pl.ds(tg * t_per_w, t_per_w), pl.ds(bg * b_per_w, b_per_w)],
        idx_v)

    lane = lax.iota(jnp.int32, _L)
    eg = [lane + (c * _L) for c in range(4)]
    i0 = [jnp.right_shift(g, 3) for g in eg]
    i2 = [jnp.bitwise_and(g, 7) for g in eg]
    zero = jnp.zeros((_L,), dtype=jnp.int32)

    def gather(k, s):
        return pltpu.make_async_copy(
            lin_hbm.at[idx_v.at[s, pl.ds(k * _VT, _VT)]],
            gbuf.at[pl.ds(k * _VT, _VT)], gsems[k])

    def out_copy(tb, s, k):
        return pltpu.make_async_copy(
            tbuf.at[tb],
            p_hbm.at[tg * t_per_w + s, :, pl.ds(bg * nbt + k, 1)],
            osems[tb])

    def transpose_block(tb, k):
        @plsc.parallel_loop(0, _VT, 1, unroll=4)
        def _(b):
            ib = jnp.full((_L,), b, dtype=jnp.int32)
            for c in range(4):
                vals = gbuf[k * _VT + b, pl.ds(c * _L, _L)] * _SCALE
                plsc.store_scatter(tbuf.at[tb], [i0[c], zero, i2[c], ib], vals)

    @pl.loop(0, t_per_w)
    def _(s):
        for k in range(8):
            gather(k, s).start()
        for k in range(8):
            gather(k, s).wait()
            tb = k % 2
            pk = (k - 2) % 8
            if k >= 2:
                out_copy(tb, s, pk).wait()
            else:
                @pl.when(s > 0)
                def _():
                    out_copy(tb, s - 1, pk).wait()
            transpose_block(tb, k)
            out_copy(tb, s, k).start()

    for k in range(6, 8):
        out_copy(k % 2, t_per_w - 1, k).wait()


def kernel(tokens, table):
    bsz, seq = tokens.shape
    vocab, emb = table.shape
    assert (vocab, emb) == (_VOCAB, _EMB) and (bsz, seq) == (4096, 200)
    t_per_w = seq // 8
    b_per_w = bsz // 4

    tokens = tokens.astype(jnp.int32)
    table = table.astype(jnp.float32)

    mesh = plsc.VectorSubcoreMesh(
        core_axis_name="c", subcore_axis_name="s",
        num_cores=_NC, num_subcores=_NS)

    lin2 = table

    tokt = tokens.T
    body = functools.partial(_gather_body, t_per_w=t_per_w, b_per_w=b_per_w)
    p = pl.kernel(
        body,
        out_type=jax.ShapeDtypeStruct((seq, 8, 32, 8, _VT), jnp.float32),
        mesh=mesh,
        compiler_params=pltpu.CompilerParams(use_tc_tiling_on_sc=False,
                                             needs_layout_passes=False),
        scratch_types=dict(
            idx_v=pltpu.VMEM((t_per_w, b_per_w), jnp.int32),
            gbuf=pltpu.VMEM((b_per_w, _EMB), jnp.float32),
            tbuf=pltpu.VMEM((2, 8, 1, 8, _VT), jnp.float32),
            gsems=[pltpu.SemaphoreType.DMA] * 8,
            osems=[pltpu.SemaphoreType.DMA] * 2,
        ),
    )(tokt, lin2)
    out = p.transpose(2, 4, 0, 1, 3).reshape(bsz, seq, emb)
    return out


# speedup vs baseline: 1.4334x; 1.1016x over previous
import functools
import math

import jax
import jax.numpy as jnp
from jax import lax
from jax.experimental import pallas as pl
from jax.experimental.pallas import tpu as pltpu
from jax.experimental.pallas import tpu_sc as plsc

_NC = 2
_NS = 16
_NW = _NC * _NS
_LANES = 16

_EMB = 64
_SCALE = math.sqrt(_EMB)

_NBUF = 4
_IDX_CHUNKS = (120, 80)


def _body(tok_hbm, table_hbm, out_hbm, idx_v, rows, gsems, osems,
          *, rows_per_w, seq, n_steps):
    wid = lax.axis_index("s") * _NC + lax.axis_index("c")
    row0 = wid * rows_per_w

    pltpu.sync_copy(tok_hbm.at[pl.ds(row0, rows_per_w)], idx_v)

    def gathers(b, r):
        cps, off = [], 0
        for w in _IDX_CHUNKS:
            cps.append(pltpu.make_async_copy(
                table_hbm.at[idx_v.at[r, pl.ds(off, w)]],
                rows.at[b, pl.ds(off, w)],
                gsems[b]))
            off += w
        return cps

    def out_copy(b, r):
        return pltpu.make_async_copy(rows.at[b], out_hbm.at[row0 + r], osems[b])

    def scale(b):
        @plsc.parallel_loop(0, seq, 1, unroll=4)
        def _(r):
            for c in range(_EMB // _LANES):
                sl = pl.ds(c * _LANES, _LANES)
                rows[b, r, sl] = rows[b, r, sl] * _SCALE

    for b in range(_NBUF):
        for cp in gathers(b, b):
            cp.start()

    @pl.loop(0, n_steps)
    def _(s):
        rb = s * _NBUF
        for b in range(_NBUF):
            @pl.when(s > 0)
            def _():
                out_copy(b, rb - _NBUF + b).wait()
                for cp in gathers(b, rb + b):
                    cp.start()
        for b in range(_NBUF):
            for cp in gathers(b, rb + b):
                cp.wait()
            scale(b)
            out_copy(b, rb + b).start()

    for b in range(_NBUF):
        out_copy(b, (n_steps - 1) * _NBUF + b).wait()


def kernel(tokens, table):
    bsz, seq = tokens.shape
    vocab, emb = table.shape
    assert emb == _EMB and seq == sum(_IDX_CHUNKS) and bsz % (_NW * _NBUF) == 0
    rows_per_w = bsz // _NW
    n_steps = rows_per_w // _NBUF

    tokens = tokens.astype(jnp.int32)
    table = table.astype(jnp.float32)

    mesh = plsc.VectorSubcoreMesh(
        core_axis_name="c", subcore_axis_name="s",
        num_cores=_NC, num_subcores=_NS)

    body = functools.partial(_body, rows_per_w=rows_per_w, seq=seq,
                             n_steps=n_steps)
    return pl.kernel(
        body,
        out_type=jax.ShapeDtypeStruct((bsz, seq, _EMB), jnp.float32),
        mesh=mesh,
        compiler_params=pltpu.CompilerParams(use_tc_tiling_on_sc=False),
        scratch_types=dict(
            idx_v=pltpu.VMEM((rows_per_w, seq), jnp.int32),
            rows=pltpu.VMEM((_NBUF, seq, _EMB), jnp.float32),
            gsems=[pltpu.SemaphoreType.DMA] * _NBUF,
            osems=[pltpu.SemaphoreType.DMA] * _NBUF,
        ),
    )(tokens, table)
